# baseline (speedup 1.0000x reference)
import jax
import jax.numpy as jnp
from jax.experimental import pallas as pl
from jax.experimental.pallas import tpu as pltpu

L = 12
SRC = 2048 + 2
TGT = 2048 + 2
D = 1024

_ROWS = L * SRC
_RB = 600


def _copy_body(enc_hbm, self_hbm, cross_hbm, out_ref, sem0, sem1, sem2):
    i = pl.program_id(0)
    rows = pl.ds(i * _RB, _RB)
    c0 = pltpu.make_async_copy(enc_hbm.at[rows], out_ref.at[0], sem0)
    c1 = pltpu.make_async_copy(self_hbm.at[rows], out_ref.at[1], sem1)
    c2 = pltpu.make_async_copy(cross_hbm.at[rows], out_ref.at[2], sem2)
    c0.start()
    c1.start()
    c2.start()
    c0.wait()
    c1.wait()
    c2.wait()


def kernel(bsz, enc_w, self_w, cross_w):
    del bsz
    enc2 = enc_w.reshape(_ROWS, D)
    self2 = self_w.reshape(_ROWS, D)
    cross2 = cross_w.reshape(_ROWS, D)
    grid = (_ROWS // _RB,)
    out = pl.pallas_call(
        _copy_body,
        grid=grid,
        in_specs=[
            pl.BlockSpec(memory_space=pl.ANY),
            pl.BlockSpec(memory_space=pl.ANY),
            pl.BlockSpec(memory_space=pl.ANY),
        ],
        out_specs=pl.BlockSpec((3, _RB, D), lambda i: (0, i, 0)),
        out_shape=jax.ShapeDtypeStruct((3, _ROWS, D), jnp.float32),
        scratch_shapes=[pltpu.SemaphoreType.DMA] * 3,
    )(enc2, self2, cross2)
    return out.reshape(3, L, SRC, D)


# speedup vs baseline: 1.0716x; 1.0716x over previous
import jax
import jax.numpy as jnp
from jax.experimental import pallas as pl
from jax.experimental.pallas import tpu as pltpu

L = 12
SRC = 2048 + 2
TGT = 2048 + 2
D = 1024

_ROWS = L * SRC
_RB = 200
_NSTEPS = _ROWS // _RB
_NBUF = 8
_A = 6


def _dma_pipeline(enc, selfw, cross, out, buf, rsem, wsem):
    srcs = (enc, selfw, cross)

    def reads(k):
        s = k % _NBUF
        rows = pl.ds(k * _RB, _RB)
        return [
            pltpu.make_async_copy(srcs[j].at[rows], buf.at[s, j], rsem.at[s, j])
            for j in range(3)
        ]

    def write(k):
        s = k % _NBUF
        return pltpu.make_async_copy(
            buf.at[s], out.at[:, pl.ds(k * _RB, _RB)], wsem.at[s]
        )

    for k in range(_A):
        for c in reads(k):
            c.start()
    for k in range(_NSTEPS):
        for c in reads(k):
            c.wait()
        write(k).start()
        kk = k + _A
        if kk < _NSTEPS:
            if kk >= _NBUF:
                write(kk - _NBUF).wait()
            for c in reads(kk):
                c.start()
    for k in range(max(0, _NSTEPS - _NBUF), _NSTEPS):
        write(k).wait()


def kernel(bsz, enc_w, self_w, cross_w):
    del bsz
    enc2 = enc_w.reshape(_ROWS, D)
    self2 = self_w.reshape(_ROWS, D)
    cross2 = cross_w.reshape(_ROWS, D)
    out = pl.pallas_call(
        _dma_pipeline,
        in_specs=[
            pl.BlockSpec(memory_space=pl.ANY),
            pl.BlockSpec(memory_space=pl.ANY),
            pl.BlockSpec(memory_space=pl.ANY),
        ],
        out_specs=pl.BlockSpec(memory_space=pl.ANY),
        out_shape=jax.ShapeDtypeStruct((3, _ROWS, D), jnp.float32),
        scratch_shapes=[
            pltpu.VMEM((_NBUF, 3, _RB, D), jnp.float32),
            pltpu.SemaphoreType.DMA((_NBUF, 3)),
            pltpu.SemaphoreType.DMA((_NBUF,)),
        ],
    )(enc2, self2, cross2)
    return out.reshape(3, L, SRC, D)
